# Initial kernel scaffold; baseline (speedup 1.0000x reference)
import jax
import jax.numpy as jnp
from jax.experimental import pallas as pl


def kernel(xyz, params):
    raise NotImplementedError("write your pallas kernel here")


# speedup vs baseline: 17.0154x; 17.0154x over previous
import functools

import jax
import jax.numpy as jnp
import numpy as np
from jax import lax
from jax.experimental import pallas as pl
from jax.experimental.pallas import tpu as pltpu
from jax.experimental.pallas import tpu_sc as plsc

_B, _N, _C = 8, 2048, 96
_HID, _HOPS, _NH, _K = 192, 3, 3, 8
_DH = _C // _NH
_BN = _B * _N
_R = 256
_NW = 32
_PW = _BN // _NW
_GP = 16
_NG = _PW // _GP


def _knn_body(x_rows_ref, x_all_ref, o_ref):
    b = pl.program_id(0)
    xr = x_rows_ref[0]
    xa = x_all_ref[0]
    sq_all = jnp.sum(xa * xa, axis=1)
    sq_r = jnp.sum(xr * xr, axis=1)
    dot = lax.dot_general(xr, xa, (((1,), (1,)), ((), ())),
                          preferred_element_type=jnp.float32)
    d2 = jnp.maximum(sq_r[:, None] + sq_all[None, :] - 2.0 * dot, 0.0)
    iota = lax.broadcasted_iota(jnp.int32, (_R, _N), 1)
    cols = []
    for _ in range(_K):
        m = jnp.min(d2, axis=1, keepdims=True)
        j = jnp.min(jnp.where(d2 == m, iota, _N), axis=1, keepdims=True)
        cols.append(j + b * _N)
        d2 = jnp.where(iota == j, jnp.inf, d2)
    o_ref[0] = jnp.concatenate(cols, axis=1)


def _knn(cur):
    return pl.pallas_call(
        _knn_body,
        grid=(_B, _N // _R),
        in_specs=[pl.BlockSpec((1, _R, _C), lambda b, r: (b, r, 0)),
                  pl.BlockSpec((1, _N, _C), lambda b, r: (b, 0, 0))],
        out_specs=pl.BlockSpec((1, _R, _K), lambda b, r: (b, r, 0)),
        out_shape=jax.ShapeDtypeStruct((_B, _N, _K), jnp.int32),
    )(cur, cur)


@functools.lru_cache(maxsize=1)
def _build_gather_mean_sc():
    mesh = plsc.VectorSubcoreMesh(core_axis_name="c", subcore_axis_name="s")

    @functools.partial(
        pl.kernel,
        mesh=mesh,
        out_type=jax.ShapeDtypeStruct((_BN, _C), jnp.float32),
        scratch_types=[
            pltpu.VMEM((_NG, 128), jnp.int32),
            pltpu.VMEM((_GP * _K, 128), jnp.float32),
            pltpu.VMEM((_GP, _C), jnp.float32),
            pltpu.SemaphoreType.DMA,
        ],
    )
    def gm(cur_hbm, gidx_hbm, out_hbm, idx_v, rows_v, acc_v, sem):
        wid = lax.axis_index("s") * 2 + lax.axis_index("c")
        pltpu.sync_copy(gidx_hbm.at[pl.ds(wid * _NG, _NG)], idx_v)

        def body(g, carry):
            pltpu.async_copy(cur_hbm.at[idx_v.at[g]], rows_v, sem).wait()
            for p in range(_GP):
                for c in range(_C // 16):
                    a = rows_v[p * _K, pl.ds(c * 16, 16)]
                    for k in range(1, _K):
                        a = a + rows_v[p * _K + k, pl.ds(c * 16, 16)]
                    acc_v[p, pl.ds(c * 16, 16)] = a
            pltpu.sync_copy(acc_v, out_hbm.at[pl.ds(wid * _PW + g * _GP, _GP)])
            return carry

        lax.fori_loop(0, _NG, body, 0)

    return gm


def _gather_mean_sc(cur_flat, gidx2):
    cur_pad = jnp.pad(cur_flat, ((0, 0), (0, 128 - _C)))
    return _build_gather_mean_sc()(cur_pad, gidx2)


def _mlp_body(x_ref, w1_ref, b1_ref, w2_ref, b2_ref, o_ref):
    x = x_ref[...] * 0.125
    h = lax.dot_general(x, w1_ref[...], (((1,), (1,)), ((), ())),
                        preferred_element_type=jnp.float32) + b1_ref[...]
    h = jnp.where(h >= 0, h, 0.2 * h)
    o_ref[...] = lax.dot_general(h, w2_ref[...], (((1,), (1,)), ((), ())),
                                 preferred_element_type=jnp.float32) + b2_ref[...]


def _mlp(nb_sum, w1, b1, w2, b2):
    tile = 512
    return pl.pallas_call(
        _mlp_body,
        grid=(_BN // tile,),
        in_specs=[pl.BlockSpec((tile, _C), lambda i: (i, 0)),
                  pl.BlockSpec((_HID, _C), lambda i: (0, 0)),
                  pl.BlockSpec((1, _HID), lambda i: (0, 0)),
                  pl.BlockSpec((_C, _HID), lambda i: (0, 0)),
                  pl.BlockSpec((1, _C), lambda i: (0, 0))],
        out_specs=pl.BlockSpec((tile, _C), lambda i: (i, 0)),
        out_shape=jax.ShapeDtypeStruct((_BN, _C), jnp.float32),
    )(nb_sum, w1, b1.reshape(1, _HID), w2, b2.reshape(1, _C))


def _mha_body(f0, f1, f2, f3, wq, wk, wv, bq, bk, bv, hm, em, ow, ob, o_ref):
    frefs = (f0, f1, f2, f3)
    T = _HOPS + 1

    def mm(a, b):
        return lax.dot_general(a, b, (((1,), (1,)), ((), ())),
                               preferred_element_type=jnp.float32)

    def mmn(a, b):
        return lax.dot_general(a, b, (((1,), (0,)), ((), ())),
                               preferred_element_type=jnp.float32)

    qs, ks, vs = [], [], []
    for t in range(T):
        x = frefs[t][...]
        qs.append(mm(x, wq[...]) + bq[...])
        ks.append(mm(x, wk[...]) + bk[...])
        vs.append(mm(x, wv[...]) + bv[...])
    scale = jnp.float32(1.0 / np.sqrt(_DH))
    acc = None
    for t in range(T):
        ss = [mmn(qs[t] * ks[s], hm[...]) * scale for s in range(T)]
        m = ss[0]
        for s in range(1, T):
            m = jnp.maximum(m, ss[s])
        es = [jnp.exp(s_ - m) for s_ in ss]
        z = es[0]
        for s in range(1, T):
            z = z + es[s]
        for s in range(T):
            w = es[s] / z
            contrib = mmn(w, em[...]) * vs[s]
            acc = contrib if acc is None else acc + contrib
    o_ref[...] = mm(acc * jnp.float32(1.0 / T), ow[...]) + ob[...]


def _mha(feats, wq, wk, wv, bq, bk, bv, hm, em, ow, ob):
    tile = 512
    wspec = [pl.BlockSpec((_C, _C), lambda i: (0, 0)),
             pl.BlockSpec((_C, _C), lambda i: (0, 0)),
             pl.BlockSpec((_C, _C), lambda i: (0, 0)),
             pl.BlockSpec((1, _C), lambda i: (0, 0)),
             pl.BlockSpec((1, _C), lambda i: (0, 0)),
             pl.BlockSpec((1, _C), lambda i: (0, 0)),
             pl.BlockSpec((_C, _NH), lambda i: (0, 0)),
             pl.BlockSpec((_NH, _C), lambda i: (0, 0)),
             pl.BlockSpec((_C, _C), lambda i: (0, 0)),
             pl.BlockSpec((1, _C), lambda i: (0, 0))]
    return pl.pallas_call(
        _mha_body,
        grid=(_BN // tile,),
        in_specs=[pl.BlockSpec((tile, _C), lambda i: (i, 0))] * 4 + wspec,
        out_specs=pl.BlockSpec((tile, _C), lambda i: (i, 0)),
        out_shape=jax.ShapeDtypeStruct((_BN, _C), jnp.float32),
    )(*feats, wq, wk, wv, bq, bk, bv, hm, em, ow, ob)


def kernel(xyz, params):
    cur = xyz
    feats = [xyz.reshape(_BN, _C)]
    for h in range(_HOPS):
        w1, b1, w2, b2 = params["hops"][h]
        gidx = _knn(cur)
        gidx2 = gidx.reshape(_BN * _K // 128, 128)
        nb_sum = _gather_mean_sc(cur.reshape(_BN, _C), gidx2)
        enc = _mlp(nb_sum, w1, b1, w2, b2)
        feats.append(enc)
        cur = enc.reshape(_B, _N, _C)
    in_w, in_b = params["in_w"], params["in_b"]
    wq, wk, wv = in_w[:_C], in_w[_C:2 * _C], in_w[2 * _C:]
    bq, bk, bv = (in_b[:_C].reshape(1, _C), in_b[_C:2 * _C].reshape(1, _C),
                  in_b[2 * _C:].reshape(1, _C))
    hm = (jnp.arange(_C)[:, None] // _DH ==
          jnp.arange(_NH)[None, :]).astype(jnp.float32)
    em = hm.T
    out = _mha(feats, wq, wk, wv, bq, bk, bv, hm, em,
               params["out_w"], params["out_b"].reshape(1, _C))
    return out.reshape(_B, _N, _C)


# speedup vs baseline: 26.7857x; 1.5742x over previous
import functools

import jax
import jax.numpy as jnp
import numpy as np
from jax import lax
from jax.experimental import pallas as pl
from jax.experimental.pallas import tpu as pltpu
from jax.experimental.pallas import tpu_sc as plsc

_B, _N, _C = 8, 2048, 96
_CP = 128
_HID, _HOPS, _NH, _K = 192, 3, 3, 8
_DH = _C // _NH
_BN = _B * _N
_R = 256
_NS = _N // 128
_NW = 32
_PW = _BN // _NW
_GP = 16
_NG = _PW // _GP

_SORT8 = [(0, 1), (2, 3), (4, 5), (6, 7),
          (0, 2), (1, 3), (4, 6), (5, 7),
          (1, 2), (5, 6),
          (0, 4), (1, 5), (2, 6), (3, 7),
          (1, 4), (3, 6),
          (2, 4), (3, 5),
          (3, 4)]
_MERGE8 = [(0, 4), (1, 5), (2, 6), (3, 7),
           (0, 2), (1, 3), (4, 6), (5, 7),
           (0, 1), (2, 3), (4, 5), (6, 7)]


def _knn_body(x_rows_ref, x_all_ref, o_ref):
    b = pl.program_id(0)
    xr = x_rows_ref[0]
    xa = x_all_ref[0]
    sq_all = jnp.sum(xa * xa, axis=1)
    sq_r = jnp.sum(xr * xr, axis=1)
    dot = lax.dot_general(xr, xa, (((1,), (1,)), ((), ())),
                          preferred_element_type=jnp.float32)
    d2 = jnp.maximum(sq_r[:, None] + sq_all[None, :] - 2.0 * dot, 0.0)
    d2i = ((lax.bitcast_convert_type(d2, jnp.int32) & jnp.int32(-16))
           + jnp.int32(1 << 26))
    slabs = [
        lax.bitcast_convert_type(
            d2i[:, s * 128:(s + 1) * 128] | jnp.int32(s), jnp.float32)
        for s in range(_NS)
    ]

    def ce(arr, i, j):
        lo = jnp.minimum(arr[i], arr[j])
        hi = jnp.maximum(arr[i], arr[j])
        arr[i], arr[j] = lo, hi

    ga = slabs[:8]
    gb = slabs[8:]
    for (i, j) in _SORT8:
        ce(ga, i, j)
        ce(gb, i, j)
    m8 = [jnp.minimum(ga[i], gb[7 - i]) for i in range(8)]
    for (i, j) in _MERGE8:
        ce(m8, i, j)

    lane_f = lax.broadcasted_iota(jnp.int32, (_R, 128), 1).astype(jnp.float32)
    cols = []
    for _ in range(_K):
        m = jnp.min(m8[0], axis=1, keepdims=True)
        eq = m8[0] == m
        lane = jnp.min(jnp.where(eq, lane_f, jnp.float32(1e30)),
                       axis=1, keepdims=True)
        lm = lane_f == lane
        for k in range(_K - 1):
            m8[k] = jnp.where(lm, m8[k + 1], m8[k])
        m8[_K - 1] = jnp.where(lm, jnp.float32(jnp.inf), m8[_K - 1])
        slot = lax.bitcast_convert_type(m, jnp.int32) & jnp.int32(15)
        j = slot * 128 + lane.astype(jnp.int32)
        cols.append(j + b * _N)
    o_ref[0] = jnp.concatenate(cols, axis=1)


def _knn(cur):
    return pl.pallas_call(
        _knn_body,
        grid=(_B, _N // _R),
        in_specs=[pl.BlockSpec((1, _R, _CP), lambda b, r: (b, r, 0)),
                  pl.BlockSpec((1, _N, _CP), lambda b, r: (b, 0, 0))],
        out_specs=pl.BlockSpec((1, _R, _K), lambda b, r: (b, r, 0)),
        out_shape=jax.ShapeDtypeStruct((_B, _N, _K), jnp.int32),
    )(cur, cur)


@functools.lru_cache(maxsize=1)
def _build_gather_mean_sc():
    mesh = plsc.VectorSubcoreMesh(core_axis_name="c", subcore_axis_name="s")

    @functools.partial(
        pl.kernel,
        mesh=mesh,
        out_type=jax.ShapeDtypeStruct((_BN, _C), jnp.float32),
        scratch_types=[
            pltpu.VMEM((_NG, 128), jnp.int32),
            pltpu.VMEM((_GP * _K, _CP), jnp.float32),
            pltpu.VMEM((_GP * _K, _CP), jnp.float32),
            pltpu.VMEM((_GP, _C), jnp.float32),
            pltpu.SemaphoreType.DMA,
            pltpu.SemaphoreType.DMA,
        ],
    )
    def gm(cur_hbm, gidx_hbm, out_hbm, idx_v, buf0, buf1, acc_v, sem0, sem1):
        wid = lax.axis_index("s") * 2 + lax.axis_index("c")
        pltpu.sync_copy(gidx_hbm.at[pl.ds(wid * _NG, _NG)], idx_v)

        def accumulate(buf, g):
            for p in range(_GP):
                for c in range(_C // 16):
                    a = buf[p * _K, pl.ds(c * 16, 16)]
                    for k in range(1, _K):
                        a = a + buf[p * _K + k, pl.ds(c * 16, 16)]
                    acc_v[p, pl.ds(c * 16, 16)] = a
            pltpu.sync_copy(acc_v, out_hbm.at[pl.ds(wid * _PW + g * _GP, _GP)])

        pltpu.async_copy(cur_hbm.at[idx_v.at[0]], buf0, sem0)

        def body(i, carry):
            g0 = 2 * i
            pltpu.async_copy(cur_hbm.at[idx_v.at[g0 + 1]], buf1, sem1)
            pltpu.make_async_copy(cur_hbm.at[idx_v.at[g0]], buf0, sem0).wait()
            accumulate(buf0, g0)

            @pl.when(i < _NG // 2 - 1)
            def _():
                pltpu.async_copy(cur_hbm.at[idx_v.at[g0 + 2]], buf0, sem0)

            pltpu.make_async_copy(
                cur_hbm.at[idx_v.at[g0 + 1]], buf1, sem1).wait()
            accumulate(buf1, g0 + 1)
            return carry

        lax.fori_loop(0, _NG // 2, body, 0)

    return gm


def _gather_mean_sc(cur_pad, gidx2):
    return _build_gather_mean_sc()(cur_pad, gidx2)


def _mlp_body(x_ref, w1_ref, b1_ref, w2_ref, b2_ref, o_ref):
    x = x_ref[...] * 0.125
    h = lax.dot_general(x, w1_ref[...], (((1,), (1,)), ((), ())),
                        preferred_element_type=jnp.float32) + b1_ref[...]
    h = jnp.where(h >= 0, h, 0.2 * h)
    o = lax.dot_general(h, w2_ref[...], (((1,), (1,)), ((), ())),
                        preferred_element_type=jnp.float32) + b2_ref[...]
    o_ref[...] = jnp.concatenate(
        [o, jnp.zeros((o.shape[0], _CP - _C), jnp.float32)], axis=1)


def _mlp(nb_sum, w1, b1, w2, b2):
    tile = 512
    return pl.pallas_call(
        _mlp_body,
        grid=(_BN // tile,),
        in_specs=[pl.BlockSpec((tile, _C), lambda i: (i, 0)),
                  pl.BlockSpec((_HID, _C), lambda i: (0, 0)),
                  pl.BlockSpec((1, _HID), lambda i: (0, 0)),
                  pl.BlockSpec((_C, _HID), lambda i: (0, 0)),
                  pl.BlockSpec((1, _C), lambda i: (0, 0))],
        out_specs=pl.BlockSpec((tile, _CP), lambda i: (i, 0)),
        out_shape=jax.ShapeDtypeStruct((_BN, _CP), jnp.float32),
    )(nb_sum, w1, b1.reshape(1, _HID), w2, b2.reshape(1, _C))


def _mha_body(f0, f1, f2, f3, wq, wk, wv, bq, bk, bv, hm, em, ow, ob, o_ref):
    frefs = (f0, f1, f2, f3)
    T = _HOPS + 1

    def mm(a, b):
        return lax.dot_general(a, b, (((1,), (1,)), ((), ())),
                               preferred_element_type=jnp.float32)

    def mmn(a, b):
        return lax.dot_general(a, b, (((1,), (0,)), ((), ())),
                               preferred_element_type=jnp.float32)

    qs, ks, vs = [], [], []
    for t in range(T):
        x = frefs[t][...]
        qs.append(mm(x, wq[...]) + bq[...])
        ks.append(mm(x, wk[...]) + bk[...])
        vs.append(mm(x, wv[...]) + bv[...])
    scale = jnp.float32(1.0 / np.sqrt(_DH))
    acc = None
    for t in range(T):
        ss = [mmn(qs[t] * ks[s], hm[...]) * scale for s in range(T)]
        m = ss[0]
        for s in range(1, T):
            m = jnp.maximum(m, ss[s])
        es = [jnp.exp(s_ - m) for s_ in ss]
        z = es[0]
        for s in range(1, T):
            z = z + es[s]
        for s in range(T):
            w = es[s] / z
            contrib = mmn(w, em[...]) * vs[s]
            acc = contrib if acc is None else acc + contrib
    o_ref[...] = mm(acc * jnp.float32(1.0 / T), ow[...]) + ob[...]


def _mha(feats, wq, wk, wv, bq, bk, bv, hm, em, ow, ob):
    tile = 512
    wspec = [pl.BlockSpec((_C, _CP), lambda i: (0, 0)),
             pl.BlockSpec((_C, _CP), lambda i: (0, 0)),
             pl.BlockSpec((_C, _CP), lambda i: (0, 0)),
             pl.BlockSpec((1, _C), lambda i: (0, 0)),
             pl.BlockSpec((1, _C), lambda i: (0, 0)),
             pl.BlockSpec((1, _C), lambda i: (0, 0)),
             pl.BlockSpec((_C, _NH), lambda i: (0, 0)),
             pl.BlockSpec((_NH, _C), lambda i: (0, 0)),
             pl.BlockSpec((_C, _C), lambda i: (0, 0)),
             pl.BlockSpec((1, _C), lambda i: (0, 0))]
    return pl.pallas_call(
        _mha_body,
        grid=(_BN // tile,),
        in_specs=[pl.BlockSpec((tile, _CP), lambda i: (i, 0))] * 4 + wspec,
        out_specs=pl.BlockSpec((tile, _C), lambda i: (i, 0)),
        out_shape=jax.ShapeDtypeStruct((_BN, _C), jnp.float32),
    )(*feats, wq, wk, wv, bq, bk, bv, hm, em, ow, ob)


def kernel(xyz, params):
    cur = jnp.pad(xyz, ((0, 0), (0, 0), (0, _CP - _C)))
    feats = [cur.reshape(_BN, _CP)]
    for h in range(_HOPS):
        w1, b1, w2, b2 = params["hops"][h]
        gidx = _knn(cur)
        gidx2 = gidx.reshape(_BN * _K // 128, 128)
        nb_sum = _gather_mean_sc(cur.reshape(_BN, _CP), gidx2)
        enc = _mlp(nb_sum, w1, b1, w2, b2)
        feats.append(enc)
        cur = enc.reshape(_B, _N, _CP)
    in_w, in_b = params["in_w"], params["in_b"]
    pad_w = lambda w: jnp.pad(w, ((0, 0), (0, _CP - _C)))
    wq, wk, wv = (pad_w(in_w[:_C]), pad_w(in_w[_C:2 * _C]),
                  pad_w(in_w[2 * _C:]))
    bq, bk, bv = (in_b[:_C].reshape(1, _C), in_b[_C:2 * _C].reshape(1, _C),
                  in_b[2 * _C:].reshape(1, _C))
    hm = (jnp.arange(_C)[:, None] // _DH ==
          jnp.arange(_NH)[None, :]).astype(jnp.float32)
    em = hm.T
    out = _mha(feats, wq, wk, wv, bq, bk, bv, hm, em,
               params["out_w"], params["out_b"].reshape(1, _C))
    return out.reshape(_B, _N, _C)
